# baseline (speedup 1.0000x reference)
import functools

import jax
import jax.numpy as jnp
from jax import lax
from jax.experimental import pallas as pl
from jax.experimental.pallas import tpu as pltpu
from jax.experimental.pallas import tpu_sc as plsc

_VERY_SMALL = 1e-8
_NSLICES = 2
_HI16 = -65536


def _pack_pairs(x):
    P, N2 = x.shape
    N = N2 // 2
    lo = lax.bitcast_convert_type(x[:, :N], jnp.int32)
    hi = lax.bitcast_convert_type(x[:, N:], jnp.int32)
    return (hi & _HI16) | lax.shift_right_logical(lo, 16)


def _pack_ab(x2, sl):
    P, N2 = x2.shape
    P2 = P // _NSLICES
    blk = 16
    nblk = P2 // blk

    def body(x_ref, o_ref):
        o_ref[...] = _pack_pairs(x_ref[...])

    return pl.pallas_call(
        body,
        grid=(nblk,),
        in_specs=[pl.BlockSpec((blk, N2), lambda i, s=sl, nb=nblk: (i + s * nb, 0))],
        out_specs=pl.BlockSpec((blk, N2 // 2), lambda i: (i, 0)),
        out_shape=jax.ShapeDtypeStruct((P2, N2 // 2), jnp.int32),
    )(x2)


def _pack_ng(g2, sl):
    P, N2 = g2.shape
    P2 = P // _NSLICES
    blk = 16
    nblk = P2 // blk

    def body(g_ref, o_ref):
        g = g_ref[...]
        ng = jnp.log(jnp.maximum(1.0 - jnp.exp(g), _VERY_SMALL))
        o_ref[...] = _pack_pairs(ng)

    return pl.pallas_call(
        body,
        grid=(nblk,),
        in_specs=[pl.BlockSpec((blk, N2), lambda i, s=sl, nb=nblk: (i + s * nb, 0))],
        out_specs=pl.BlockSpec((blk, N2 // 2), lambda i: (i, 0)),
        out_shape=jax.ShapeDtypeStruct((P2, N2 // 2), jnp.int32),
    )(g2)


def _pack_idx(idx):
    r = idx.astype(jnp.int32).reshape(-1, 2, 16)
    return (r[:, 0, :] | (r[:, 1, :] << 16)).reshape(-1)


@functools.cache
def _sc_gather(B, N, K, sl):
    NC, NS = 2, 16
    NW = NC * NS
    NPAIR = B // 2 // _NSLICES
    HP = sl * NPAIR
    PPT = NPAIR // NW
    OCH = 2048
    NCH = K // OCH
    GRP = OCH // 32
    KP = K // 2
    assert NPAIR % NW == 0 and PPT % 2 == 0 and K % OCH == 0 and K % 32 == 0

    mesh = plsc.VectorSubcoreMesh(core_axis_name="c", subcore_axis_name="s")

    @functools.partial(
        pl.kernel,
        mesh=mesh,
        compiler_params=pltpu.CompilerParams(needs_layout_passes=False),
        out_type=(),
        scratch_types=[
            pltpu.VMEM((KP,), jnp.int32),
            pltpu.VMEM((KP,), jnp.int32),
            pltpu.VMEM((KP,), jnp.int32),
            pltpu.VMEM((N,), jnp.int32),
            pltpu.VMEM((N,), jnp.int32),
            pltpu.VMEM((N,), jnp.int32),
            pltpu.VMEM((N,), jnp.int32),
            pltpu.VMEM((OCH,), jnp.float32),
            pltpu.VMEM((OCH,), jnp.float32),
            pltpu.VMEM((OCH,), jnp.float32),
            pltpu.VMEM((OCH,), jnp.float32),
            pltpu.SemaphoreType.DMA,
            pltpu.SemaphoreType.DMA,
            pltpu.SemaphoreType.DMA,
            pltpu.SemaphoreType.DMA,
        ],
    )
    def sc(ta_hbm, tb_hbm, tg_hbm, pai_hbm, pbi_hbm, pgi_hbm, out_hbm,
           pai, pbi, pgi, ta0, ta1, tb, tg, oe0, oo0, oe1, oo1,
           sem_in, sem_a, sem_o0, sem_o1):
        wid = lax.axis_index("s") * NC + lax.axis_index("c")
        pltpu.sync_copy(pai_hbm, pai)
        pltpu.sync_copy(pbi_hbm, pbi)
        pltpu.sync_copy(pgi_hbm, pgi)
        pbase = wid * PPT
        m16 = jnp.int32(0xFFFF)

        def wait_a(buf):
            pltpu.make_async_copy(ta_hbm.at[0], buf, sem_a).wait()

        def wait_bg():
            pltpu.make_async_copy(tb_hbm.at[0], tb, sem_in).wait()
            pltpu.make_async_copy(tg_hbm.at[0], tg, sem_in).wait()

        def load_bg(p):
            pltpu.async_copy(tb_hbm.at[HP + p], tb, sem_in)
            pltpu.async_copy(tg_hbm.at[HP + p], tg, sem_in)

        def do_chunk(c, oe, oo, ta):
            wbase = c * (OCH // 2)

            @plsc.parallel_loop(0, GRP, unroll=4)
            def grp(g):
                w = wbase + g * 16
                wa = pai[pl.ds(w, 16)]
                wb = pbi[pl.ds(w, 16)]
                wg = pgi[pl.ds(w, 16)]
                ga_l = plsc.load_gather(ta, [lax.bitwise_and(wa, m16)])
                ga_h = plsc.load_gather(ta, [lax.shift_right_logical(wa, 16)])
                gb_l = plsc.load_gather(tb, [lax.bitwise_and(wb, m16)])
                gb_h = plsc.load_gather(tb, [lax.shift_right_logical(wb, 16)])
                gg_l = plsc.load_gather(tg, [lax.bitwise_and(wg, m16)])
                gg_h = plsc.load_gather(tg, [lax.shift_right_logical(wg, 16)])
                ae_l = plsc.bitcast(lax.shift_left(ga_l, 16), jnp.float32)
                ae_h = plsc.bitcast(lax.shift_left(ga_h, 16), jnp.float32)
                be_l = plsc.bitcast(lax.shift_left(gb_l, 16), jnp.float32)
                be_h = plsc.bitcast(lax.shift_left(gb_h, 16), jnp.float32)
                ge_l = plsc.bitcast(lax.shift_left(gg_l, 16), jnp.float32)
                ge_h = plsc.bitcast(lax.shift_left(gg_h, 16), jnp.float32)
                ao_l = plsc.bitcast(ga_l, jnp.float32)
                ao_h = plsc.bitcast(ga_h, jnp.float32)
                bo_l = plsc.bitcast(gb_l, jnp.float32)
                bo_h = plsc.bitcast(gb_h, jnp.float32)
                go_l = plsc.bitcast(gg_l, jnp.float32)
                go_h = plsc.bitcast(gg_h, jnp.float32)
                o = g * 32
                oe[pl.ds(o, 16)] = jnp.maximum(ae_l + be_l - ge_l, 0.0)
                oe[pl.ds(o + 16, 16)] = jnp.maximum(ae_h + be_h - ge_h, 0.0)
                oo[pl.ds(o, 16)] = jnp.maximum(ao_l + bo_l - go_l, 0.0)
                oo[pl.ds(o + 16, 16)] = jnp.maximum(ao_h + bo_h - go_h, 0.0)

        def do_pair(p, ta, pending):
            row = 2 * (HP + p)
            for c in range(NCH):
                oe, oo, slot, sem = ((oe0, oo0, 0, sem_o0) if c % 2 == 0
                                     else (oe1, oo1, 1, sem_o1))
                if pending[slot] is not None:
                    pending[slot][0].wait()
                    pending[slot][1].wait()
                do_chunk(c, oe, oo, ta)
                he = pltpu.async_copy(oe, out_hbm.at[row, pl.ds(c * OCH, OCH)], sem)
                ho = pltpu.async_copy(oo, out_hbm.at[row + 1, pl.ds(c * OCH, OCH)], sem)
                pending[slot] = (he, ho)

        pltpu.async_copy(ta_hbm.at[HP + pbase], ta0, sem_a)
        load_bg(pbase)

        def pairstep(i, carry):
            p0 = pbase + 2 * i
            pending = [None, None]
            pltpu.async_copy(ta_hbm.at[HP + p0 + 1], ta1, sem_a)
            wait_a(ta0)
            wait_bg()
            do_pair(p0, ta0, pending)
            nxt = jnp.minimum(p0 + 2, pbase + PPT - 1)
            pltpu.async_copy(ta_hbm.at[HP + nxt], ta0, sem_a)
            load_bg(p0 + 1)
            wait_a(ta1)
            wait_bg()
            do_pair(p0 + 1, ta1, pending)
            load_bg(nxt)
            pending[0][0].wait()
            pending[0][1].wait()
            pending[1][0].wait()
            pending[1][1].wait()
            return carry

        lax.fori_loop(0, PPT // 2, pairstep, 0)
        wait_a(ta0)
        wait_bg()

    return sc


def kernel(alpha, beta, gamma, alpha_idx, beta_idx, gamma_idx):
    B, N = alpha.shape
    K = alpha_idx.shape[0]
    pai = _pack_idx(alpha_idx)
    pbi = _pack_idx(beta_idx)
    pgi = _pack_idx(gamma_idx)
    a2 = alpha.reshape(B // 2, 2 * N)
    b2 = beta.reshape(B // 2, 2 * N)
    g2 = gamma.reshape(B // 2, 2 * N)
    out_ref = jax.new_ref(jax.lax.empty((B, K), jnp.float32))
    for sl in range(_NSLICES):
        ta = _pack_ab(a2, sl)
        tb = _pack_ab(b2, sl)
        tg = _pack_ng(g2, sl)
        _sc_gather(B, N, K, sl)(ta, tb, tg, pai, pbi, pgi, out_ref)
    return jax.freeze(out_ref)


# speedup vs baseline: 1.0006x; 1.0006x over previous
import functools

import jax
import jax.numpy as jnp
from jax import lax
from jax.experimental import pallas as pl
from jax.experimental.pallas import tpu as pltpu
from jax.experimental.pallas import tpu_sc as plsc

_VERY_SMALL = 1e-8
_NSLICES = 2
_HI16 = -65536


def _pack_pairs(x):
    P, N2 = x.shape
    N = N2 // 2
    lo = lax.bitcast_convert_type(x[:, :N], jnp.int32)
    hi = lax.bitcast_convert_type(x[:, N:], jnp.int32)
    return (hi & _HI16) | lax.shift_right_logical(lo, 16)


def _pack_ab(x2, sl):
    P, N2 = x2.shape
    P2 = P // _NSLICES
    blk = 16
    nblk = P2 // blk

    def body(x_ref, o_ref):
        o_ref[...] = _pack_pairs(x_ref[...])

    return pl.pallas_call(
        body,
        grid=(nblk,),
        in_specs=[pl.BlockSpec((blk, N2), lambda i, s=sl, nb=nblk: (i + s * nb, 0))],
        out_specs=pl.BlockSpec((blk, N2 // 2), lambda i: (i, 0)),
        out_shape=jax.ShapeDtypeStruct((P2, N2 // 2), jnp.int32),
    )(x2)


def _pack_ng(g2, sl):
    P, N2 = g2.shape
    P2 = P // _NSLICES
    blk = 16
    nblk = P2 // blk

    def body(g_ref, o_ref):
        g = g_ref[...]
        ng = jnp.log(jnp.maximum(1.0 - jnp.exp(g), _VERY_SMALL))
        o_ref[...] = _pack_pairs(ng)

    return pl.pallas_call(
        body,
        grid=(nblk,),
        in_specs=[pl.BlockSpec((blk, N2), lambda i, s=sl, nb=nblk: (i + s * nb, 0))],
        out_specs=pl.BlockSpec((blk, N2 // 2), lambda i: (i, 0)),
        out_shape=jax.ShapeDtypeStruct((P2, N2 // 2), jnp.int32),
    )(g2)


def _pack_idx(idx):
    r = idx.astype(jnp.int32).reshape(-1, 2, 16)
    return (r[:, 0, :] | (r[:, 1, :] << 16)).reshape(-1)


@functools.cache
def _sc_gather(B, N, K, sl):
    NC, NS = 2, 16
    NW = NC * NS
    NPAIR = B // 2 // _NSLICES
    HP = sl * NPAIR
    PPT = NPAIR // NW
    OCH = 2048
    NCH = K // OCH
    GRP = OCH // 32
    KP = K // 2
    assert NPAIR % NW == 0 and PPT % 2 == 0 and K % OCH == 0 and K % 32 == 0

    mesh = plsc.VectorSubcoreMesh(core_axis_name="c", subcore_axis_name="s")

    @functools.partial(
        pl.kernel,
        mesh=mesh,
        compiler_params=pltpu.CompilerParams(needs_layout_passes=False),
        out_type=(),
        scratch_types=[
            pltpu.VMEM((KP,), jnp.int32),
            pltpu.VMEM((KP,), jnp.int32),
            pltpu.VMEM((KP,), jnp.int32),
            pltpu.VMEM((N,), jnp.int32),
            pltpu.VMEM((N,), jnp.int32),
            pltpu.VMEM((N,), jnp.int32),
            pltpu.VMEM((N,), jnp.int32),
            pltpu.VMEM((OCH,), jnp.float32),
            pltpu.VMEM((OCH,), jnp.float32),
            pltpu.VMEM((OCH,), jnp.float32),
            pltpu.VMEM((OCH,), jnp.float32),
            pltpu.SemaphoreType.DMA,
            pltpu.SemaphoreType.DMA,
            pltpu.SemaphoreType.DMA,
            pltpu.SemaphoreType.DMA,
        ],
    )
    def sc(ta_hbm, tb_hbm, tg_hbm, pai_hbm, pbi_hbm, pgi_hbm, out_hbm,
           pai, pbi, pgi, ta0, ta1, tb, tg, oe0, oo0, oe1, oo1,
           sem_in, sem_a, sem_o0, sem_o1):
        wid = lax.axis_index("s") * NC + lax.axis_index("c")
        pltpu.sync_copy(pai_hbm, pai)
        pltpu.sync_copy(pbi_hbm, pbi)
        pltpu.sync_copy(pgi_hbm, pgi)
        pbase = wid * PPT
        m16 = jnp.int32(0xFFFF)

        def wait_a(buf):
            pltpu.make_async_copy(ta_hbm.at[0], buf, sem_a).wait()

        def wait_bg():
            pltpu.make_async_copy(tb_hbm.at[0], tb, sem_in).wait()
            pltpu.make_async_copy(tg_hbm.at[0], tg, sem_in).wait()

        def load_bg(p):
            pltpu.async_copy(tb_hbm.at[p], tb, sem_in)
            pltpu.async_copy(tg_hbm.at[p], tg, sem_in)

        def do_chunk(c, oe, oo, ta):
            wbase = c * (OCH // 2)

            @plsc.parallel_loop(0, GRP, unroll=4)
            def grp(g):
                w = wbase + g * 16
                wa = pai[pl.ds(w, 16)]
                wb = pbi[pl.ds(w, 16)]
                wg = pgi[pl.ds(w, 16)]
                ga_l = plsc.load_gather(ta, [lax.bitwise_and(wa, m16)])
                ga_h = plsc.load_gather(ta, [lax.shift_right_logical(wa, 16)])
                gb_l = plsc.load_gather(tb, [lax.bitwise_and(wb, m16)])
                gb_h = plsc.load_gather(tb, [lax.shift_right_logical(wb, 16)])
                gg_l = plsc.load_gather(tg, [lax.bitwise_and(wg, m16)])
                gg_h = plsc.load_gather(tg, [lax.shift_right_logical(wg, 16)])
                ae_l = plsc.bitcast(lax.shift_left(ga_l, 16), jnp.float32)
                ae_h = plsc.bitcast(lax.shift_left(ga_h, 16), jnp.float32)
                be_l = plsc.bitcast(lax.shift_left(gb_l, 16), jnp.float32)
                be_h = plsc.bitcast(lax.shift_left(gb_h, 16), jnp.float32)
                ge_l = plsc.bitcast(lax.shift_left(gg_l, 16), jnp.float32)
                ge_h = plsc.bitcast(lax.shift_left(gg_h, 16), jnp.float32)
                ao_l = plsc.bitcast(ga_l, jnp.float32)
                ao_h = plsc.bitcast(ga_h, jnp.float32)
                bo_l = plsc.bitcast(gb_l, jnp.float32)
                bo_h = plsc.bitcast(gb_h, jnp.float32)
                go_l = plsc.bitcast(gg_l, jnp.float32)
                go_h = plsc.bitcast(gg_h, jnp.float32)
                o = g * 32
                oe[pl.ds(o, 16)] = jnp.maximum(ae_l + be_l - ge_l, 0.0)
                oe[pl.ds(o + 16, 16)] = jnp.maximum(ae_h + be_h - ge_h, 0.0)
                oo[pl.ds(o, 16)] = jnp.maximum(ao_l + bo_l - go_l, 0.0)
                oo[pl.ds(o + 16, 16)] = jnp.maximum(ao_h + bo_h - go_h, 0.0)

        def do_pair(p, ta, pending):
            row = 2 * (HP + p)
            for c in range(NCH):
                oe, oo, slot, sem = ((oe0, oo0, 0, sem_o0) if c % 2 == 0
                                     else (oe1, oo1, 1, sem_o1))
                if pending[slot] is not None:
                    pending[slot][0].wait()
                    pending[slot][1].wait()
                do_chunk(c, oe, oo, ta)
                he = pltpu.async_copy(oe, out_hbm.at[row, pl.ds(c * OCH, OCH)], sem)
                ho = pltpu.async_copy(oo, out_hbm.at[row + 1, pl.ds(c * OCH, OCH)], sem)
                pending[slot] = (he, ho)

        pltpu.async_copy(ta_hbm.at[pbase], ta0, sem_a)
        load_bg(pbase)

        def pairstep(i, carry):
            p0 = pbase + 2 * i
            pending = [None, None]
            pltpu.async_copy(ta_hbm.at[p0 + 1], ta1, sem_a)
            wait_a(ta0)
            wait_bg()
            do_pair(p0, ta0, pending)
            nxt = jnp.minimum(p0 + 2, pbase + PPT - 1)
            pltpu.async_copy(ta_hbm.at[nxt], ta0, sem_a)
            load_bg(p0 + 1)
            wait_a(ta1)
            wait_bg()
            do_pair(p0 + 1, ta1, pending)
            load_bg(nxt)
            pending[0][0].wait()
            pending[0][1].wait()
            pending[1][0].wait()
            pending[1][1].wait()
            return carry

        lax.fori_loop(0, PPT // 2, pairstep, 0)
        wait_a(ta0)
        wait_bg()

    return sc


def kernel(alpha, beta, gamma, alpha_idx, beta_idx, gamma_idx):
    B, N = alpha.shape
    K = alpha_idx.shape[0]
    pai = _pack_idx(alpha_idx)
    pbi = _pack_idx(beta_idx)
    pgi = _pack_idx(gamma_idx)
    a2 = alpha.reshape(B // 2, 2 * N)
    b2 = beta.reshape(B // 2, 2 * N)
    g2 = gamma.reshape(B // 2, 2 * N)
    out_ref = jax.new_ref(jax.lax.empty((B, K), jnp.float32))
    for sl in range(_NSLICES):
        ta = _pack_ab(a2, sl)
        tb = _pack_ab(b2, sl)
        tg = _pack_ng(g2, sl)
        _sc_gather(B, N, K, sl)(ta, tb, tg, pai, pbi, pgi, out_ref)
    return jax.freeze(out_ref)


# speedup vs baseline: 1.0253x; 1.0246x over previous
import functools

import jax
import jax.numpy as jnp
from jax import lax
from jax.experimental import pallas as pl
from jax.experimental.pallas import tpu as pltpu
from jax.experimental.pallas import tpu_sc as plsc

_VERY_SMALL = 1e-8
_NSLICES = 2
_HI16 = -65536


def _pack_pairs(x):
    P, N2 = x.shape
    N = N2 // 2
    lo = lax.bitcast_convert_type(x[:, :N], jnp.int32)
    hi = lax.bitcast_convert_type(x[:, N:], jnp.int32)
    return (hi & _HI16) | lax.shift_right_logical(lo, 16)


def _pack_all(a2, b2, g2, sl):
    P, N2 = a2.shape
    P2 = P // _NSLICES
    blk = 16
    nblk = P2 // blk

    def body(a_ref, b_ref, g_ref, oa_ref, ob_ref, og_ref):
        oa_ref[...] = _pack_pairs(a_ref[...])
        ob_ref[...] = _pack_pairs(b_ref[...])
        g = g_ref[...]
        ng = jnp.log(jnp.maximum(1.0 - jnp.exp(g), _VERY_SMALL))
        og_ref[...] = _pack_pairs(ng)

    ispec = pl.BlockSpec((blk, N2), lambda i, s=sl, nb=nblk: (i + s * nb, 0))
    ospec = pl.BlockSpec((blk, N2 // 2), lambda i: (i, 0))
    oshape = jax.ShapeDtypeStruct((P2, N2 // 2), jnp.int32)
    return pl.pallas_call(
        body,
        grid=(nblk,),
        in_specs=[ispec, ispec, ispec],
        out_specs=[ospec, ospec, ospec],
        out_shape=[oshape, oshape, oshape],
    )(a2, b2, g2)


def _pack_idx(idx):
    r = idx.astype(jnp.int32).reshape(-1, 2, 16)
    return (r[:, 0, :] | (r[:, 1, :] << 16)).reshape(-1)


@functools.cache
def _sc_gather(B, N, K, sl):
    NC, NS = 2, 16
    NW = NC * NS
    NPAIR = B // 2 // _NSLICES
    HP = sl * NPAIR
    PPT = NPAIR // NW
    OCH = 2048
    NCH = K // OCH
    GRP = OCH // 32
    KP = K // 2
    assert NPAIR % NW == 0 and PPT % 2 == 0 and K % OCH == 0 and K % 32 == 0

    mesh = plsc.VectorSubcoreMesh(core_axis_name="c", subcore_axis_name="s")

    @functools.partial(
        pl.kernel,
        mesh=mesh,
        compiler_params=pltpu.CompilerParams(needs_layout_passes=False),
        out_type=(),
        scratch_types=[
            pltpu.VMEM((KP,), jnp.int32),
            pltpu.VMEM((KP,), jnp.int32),
            pltpu.VMEM((KP,), jnp.int32),
            pltpu.VMEM((N,), jnp.int32),
            pltpu.VMEM((N,), jnp.int32),
            pltpu.VMEM((N,), jnp.int32),
            pltpu.VMEM((N,), jnp.int32),
            pltpu.VMEM((OCH,), jnp.float32),
            pltpu.VMEM((OCH,), jnp.float32),
            pltpu.VMEM((OCH,), jnp.float32),
            pltpu.VMEM((OCH,), jnp.float32),
            pltpu.SemaphoreType.DMA,
            pltpu.SemaphoreType.DMA,
            pltpu.SemaphoreType.DMA,
            pltpu.SemaphoreType.DMA,
        ],
    )
    def sc(ta_hbm, tb_hbm, tg_hbm, pai_hbm, pbi_hbm, pgi_hbm, out_hbm,
           pai, pbi, pgi, ta0, ta1, tb, tg, oe0, oo0, oe1, oo1,
           sem_in, sem_a, sem_o0, sem_o1):
        wid = lax.axis_index("s") * NC + lax.axis_index("c")
        pltpu.sync_copy(pai_hbm, pai)
        pltpu.sync_copy(pbi_hbm, pbi)
        pltpu.sync_copy(pgi_hbm, pgi)
        pbase = wid * PPT
        m16 = jnp.int32(0xFFFF)

        def wait_a(buf):
            pltpu.make_async_copy(ta_hbm.at[0], buf, sem_a).wait()

        def wait_bg():
            pltpu.make_async_copy(tb_hbm.at[0], tb, sem_in).wait()
            pltpu.make_async_copy(tg_hbm.at[0], tg, sem_in).wait()

        def load_bg(p):
            pltpu.async_copy(tb_hbm.at[p], tb, sem_in)
            pltpu.async_copy(tg_hbm.at[p], tg, sem_in)

        def do_chunk(c, oe, oo, ta):
            wbase = c * (OCH // 2)

            @plsc.parallel_loop(0, GRP, unroll=4)
            def grp(g):
                w = wbase + g * 16
                wa = pai[pl.ds(w, 16)]
                wb = pbi[pl.ds(w, 16)]
                wg = pgi[pl.ds(w, 16)]
                ga_l = plsc.load_gather(ta, [lax.bitwise_and(wa, m16)])
                ga_h = plsc.load_gather(ta, [lax.shift_right_logical(wa, 16)])
                gb_l = plsc.load_gather(tb, [lax.bitwise_and(wb, m16)])
                gb_h = plsc.load_gather(tb, [lax.shift_right_logical(wb, 16)])
                gg_l = plsc.load_gather(tg, [lax.bitwise_and(wg, m16)])
                gg_h = plsc.load_gather(tg, [lax.shift_right_logical(wg, 16)])
                ae_l = plsc.bitcast(lax.shift_left(ga_l, 16), jnp.float32)
                ae_h = plsc.bitcast(lax.shift_left(ga_h, 16), jnp.float32)
                be_l = plsc.bitcast(lax.shift_left(gb_l, 16), jnp.float32)
                be_h = plsc.bitcast(lax.shift_left(gb_h, 16), jnp.float32)
                ge_l = plsc.bitcast(lax.shift_left(gg_l, 16), jnp.float32)
                ge_h = plsc.bitcast(lax.shift_left(gg_h, 16), jnp.float32)
                ao_l = plsc.bitcast(ga_l, jnp.float32)
                ao_h = plsc.bitcast(ga_h, jnp.float32)
                bo_l = plsc.bitcast(gb_l, jnp.float32)
                bo_h = plsc.bitcast(gb_h, jnp.float32)
                go_l = plsc.bitcast(gg_l, jnp.float32)
                go_h = plsc.bitcast(gg_h, jnp.float32)
                o = g * 32
                oe[pl.ds(o, 16)] = jnp.maximum(ae_l + be_l - ge_l, 0.0)
                oe[pl.ds(o + 16, 16)] = jnp.maximum(ae_h + be_h - ge_h, 0.0)
                oo[pl.ds(o, 16)] = jnp.maximum(ao_l + bo_l - go_l, 0.0)
                oo[pl.ds(o + 16, 16)] = jnp.maximum(ao_h + bo_h - go_h, 0.0)

        def do_pair(p, ta, pending):
            row = 2 * (HP + p)
            for c in range(NCH):
                oe, oo, slot, sem = ((oe0, oo0, 0, sem_o0) if c % 2 == 0
                                     else (oe1, oo1, 1, sem_o1))
                if pending[slot] is not None:
                    pending[slot][0].wait()
                    pending[slot][1].wait()
                do_chunk(c, oe, oo, ta)
                he = pltpu.async_copy(oe, out_hbm.at[row, pl.ds(c * OCH, OCH)], sem)
                ho = pltpu.async_copy(oo, out_hbm.at[row + 1, pl.ds(c * OCH, OCH)], sem)
                pending[slot] = (he, ho)

        pltpu.async_copy(ta_hbm.at[pbase], ta0, sem_a)
        load_bg(pbase)

        def pairstep(i, carry):
            p0 = pbase + 2 * i
            pending = [None, None]
            pltpu.async_copy(ta_hbm.at[p0 + 1], ta1, sem_a)
            wait_a(ta0)
            wait_bg()
            do_pair(p0, ta0, pending)
            nxt = jnp.minimum(p0 + 2, pbase + PPT - 1)
            pltpu.async_copy(ta_hbm.at[nxt], ta0, sem_a)
            load_bg(p0 + 1)
            wait_a(ta1)
            wait_bg()
            do_pair(p0 + 1, ta1, pending)
            load_bg(nxt)
            pending[0][0].wait()
            pending[0][1].wait()
            pending[1][0].wait()
            pending[1][1].wait()
            return carry

        lax.fori_loop(0, PPT // 2, pairstep, 0)
        wait_a(ta0)
        wait_bg()

    return sc


def kernel(alpha, beta, gamma, alpha_idx, beta_idx, gamma_idx):
    B, N = alpha.shape
    K = alpha_idx.shape[0]
    pai = _pack_idx(alpha_idx)
    pbi = _pack_idx(beta_idx)
    pgi = _pack_idx(gamma_idx)
    a2 = alpha.reshape(B // 2, 2 * N)
    b2 = beta.reshape(B // 2, 2 * N)
    g2 = gamma.reshape(B // 2, 2 * N)
    out_ref = jax.new_ref(jax.lax.empty((B, K), jnp.float32))
    for sl in range(_NSLICES):
        ta, tb, tg = _pack_all(a2, b2, g2, sl)
        _sc_gather(B, N, K, sl)(ta, tb, tg, pai, pbi, pgi, out_ref)
    return jax.freeze(out_ref)


# speedup vs baseline: 1.0270x; 1.0017x over previous
import functools

import jax
import jax.numpy as jnp
from jax import lax
from jax.experimental import pallas as pl
from jax.experimental.pallas import tpu as pltpu
from jax.experimental.pallas import tpu_sc as plsc

_VERY_SMALL = 1e-8
_NSLICES = 2
_HI16 = -65536


def _pack_pairs(x):
    P, N2 = x.shape
    N = N2 // 2
    lo = lax.bitcast_convert_type(x[:, :N], jnp.int32)
    hi = lax.bitcast_convert_type(x[:, N:], jnp.int32)
    return (hi & _HI16) | lax.shift_right_logical(lo, 16)


def _pack_all(a2, b2, g2, sl):
    P, N2 = a2.shape
    P2 = P // _NSLICES
    blk = 16
    nblk = P2 // blk

    def body(a_ref, b_ref, g_ref, oa_ref, ob_ref, og_ref):
        f32 = jnp.float32
        oa_ref[...] = lax.bitcast_convert_type(_pack_pairs(a_ref[...]), f32)
        ob_ref[...] = lax.bitcast_convert_type(_pack_pairs(b_ref[...]), f32)
        g = g_ref[...]
        ng = jnp.log(jnp.maximum(1.0 - jnp.exp(g), _VERY_SMALL))
        og_ref[...] = lax.bitcast_convert_type(_pack_pairs(ng), f32)

    ispec = pl.BlockSpec((blk, N2), lambda i, s=sl, nb=nblk: (i + s * nb, 0))
    ospec = pl.BlockSpec((blk, N2 // 2), lambda i: (i, 0))
    oshape = jax.ShapeDtypeStruct((P2, N2 // 2), jnp.float32)
    return pl.pallas_call(
        body,
        grid=(nblk,),
        in_specs=[ispec, ispec, ispec],
        out_specs=[ospec, ospec, ospec],
        out_shape=[oshape, oshape, oshape],
    )(a2, b2, g2)


def _pack_idx(idx):
    r = idx.astype(jnp.int32).reshape(-1, 2, 16)
    return (r[:, 0, :] | (r[:, 1, :] << 16)).reshape(-1)


@functools.cache
def _sc_gather(B, N, K, sl):
    NC, NS = 2, 16
    NW = NC * NS
    NPAIR = B // 2 // _NSLICES
    HP = sl * NPAIR
    PPT = NPAIR // NW
    OCH = 2048
    NCH = K // OCH
    GRP = OCH // 32
    KP = K // 2
    assert NPAIR % NW == 0 and PPT % 2 == 0 and K % OCH == 0 and K % 32 == 0

    mesh = plsc.VectorSubcoreMesh(core_axis_name="c", subcore_axis_name="s")

    @functools.partial(
        pl.kernel,
        mesh=mesh,
        compiler_params=pltpu.CompilerParams(needs_layout_passes=False),
        out_type=(),
        scratch_types=[
            pltpu.VMEM((KP,), jnp.int32),
            pltpu.VMEM((KP,), jnp.int32),
            pltpu.VMEM((KP,), jnp.int32),
            pltpu.VMEM((N,), jnp.float32),
            pltpu.VMEM((N,), jnp.float32),
            pltpu.VMEM((N,), jnp.float32),
            pltpu.VMEM((N,), jnp.float32),
            pltpu.VMEM((OCH,), jnp.float32),
            pltpu.VMEM((OCH,), jnp.float32),
            pltpu.VMEM((OCH,), jnp.float32),
            pltpu.VMEM((OCH,), jnp.float32),
            pltpu.SemaphoreType.DMA,
            pltpu.SemaphoreType.DMA,
            pltpu.SemaphoreType.DMA,
            pltpu.SemaphoreType.DMA,
        ],
    )
    def sc(ta_hbm, tb_hbm, tg_hbm, pai_hbm, pbi_hbm, pgi_hbm, out_hbm,
           pai, pbi, pgi, ta0, ta1, tb, tg, oe0, oo0, oe1, oo1,
           sem_in, sem_a, sem_o0, sem_o1):
        wid = lax.axis_index("s") * NC + lax.axis_index("c")
        pltpu.sync_copy(pai_hbm, pai)
        pltpu.sync_copy(pbi_hbm, pbi)
        pltpu.sync_copy(pgi_hbm, pgi)
        pbase = wid * PPT
        m16 = jnp.int32(0xFFFF)

        def wait_a(buf):
            pltpu.make_async_copy(ta_hbm.at[0], buf, sem_a).wait()

        def wait_bg():
            pltpu.make_async_copy(tb_hbm.at[0], tb, sem_in).wait()
            pltpu.make_async_copy(tg_hbm.at[0], tg, sem_in).wait()

        def load_bg(p):
            pltpu.async_copy(tb_hbm.at[p], tb, sem_in)
            pltpu.async_copy(tg_hbm.at[p], tg, sem_in)

        def do_chunk(c, oe, oo, ta):
            wbase = c * (OCH // 2)

            @plsc.parallel_loop(0, GRP, unroll=4)
            def grp(g):
                w = wbase + g * 16
                wa = pai[pl.ds(w, 16)]
                wb = pbi[pl.ds(w, 16)]
                wg = pgi[pl.ds(w, 16)]
                i32 = jnp.int32
                f32 = jnp.float32
                ga_l = plsc.bitcast(plsc.load_gather(ta, [lax.bitwise_and(wa, m16)]), i32)
                ga_h = plsc.bitcast(plsc.load_gather(ta, [lax.shift_right_logical(wa, 16)]), i32)
                gb_l = plsc.bitcast(plsc.load_gather(tb, [lax.bitwise_and(wb, m16)]), i32)
                gb_h = plsc.bitcast(plsc.load_gather(tb, [lax.shift_right_logical(wb, 16)]), i32)
                gg_l = plsc.bitcast(plsc.load_gather(tg, [lax.bitwise_and(wg, m16)]), i32)
                gg_h = plsc.bitcast(plsc.load_gather(tg, [lax.shift_right_logical(wg, 16)]), i32)
                ae_l = plsc.bitcast(lax.shift_left(ga_l, 16), f32)
                ae_h = plsc.bitcast(lax.shift_left(ga_h, 16), f32)
                be_l = plsc.bitcast(lax.shift_left(gb_l, 16), f32)
                be_h = plsc.bitcast(lax.shift_left(gb_h, 16), f32)
                ge_l = plsc.bitcast(lax.shift_left(gg_l, 16), f32)
                ge_h = plsc.bitcast(lax.shift_left(gg_h, 16), f32)
                ao_l = plsc.bitcast(ga_l, f32)
                ao_h = plsc.bitcast(ga_h, f32)
                bo_l = plsc.bitcast(gb_l, f32)
                bo_h = plsc.bitcast(gb_h, f32)
                go_l = plsc.bitcast(gg_l, f32)
                go_h = plsc.bitcast(gg_h, f32)
                o = g * 32
                oe[pl.ds(o, 16)] = jnp.maximum(ae_l + be_l - ge_l, 0.0)
                oe[pl.ds(o + 16, 16)] = jnp.maximum(ae_h + be_h - ge_h, 0.0)
                oo[pl.ds(o, 16)] = jnp.maximum(ao_l + bo_l - go_l, 0.0)
                oo[pl.ds(o + 16, 16)] = jnp.maximum(ao_h + bo_h - go_h, 0.0)

        def do_pair(p, ta, pending):
            row = 2 * (HP + p)
            for c in range(NCH):
                oe, oo, slot, sem = ((oe0, oo0, 0, sem_o0) if c % 2 == 0
                                     else (oe1, oo1, 1, sem_o1))
                if pending[slot] is not None:
                    pending[slot][0].wait()
                    pending[slot][1].wait()
                do_chunk(c, oe, oo, ta)
                he = pltpu.async_copy(oe, out_hbm.at[row, pl.ds(c * OCH, OCH)], sem)
                ho = pltpu.async_copy(oo, out_hbm.at[row + 1, pl.ds(c * OCH, OCH)], sem)
                pending[slot] = (he, ho)

        pltpu.async_copy(ta_hbm.at[pbase], ta0, sem_a)
        load_bg(pbase)

        def pairstep(i, carry):
            p0 = pbase + 2 * i
            pending = [None, None]
            pltpu.async_copy(ta_hbm.at[p0 + 1], ta1, sem_a)
            wait_a(ta0)
            wait_bg()
            do_pair(p0, ta0, pending)
            nxt = jnp.minimum(p0 + 2, pbase + PPT - 1)
            pltpu.async_copy(ta_hbm.at[nxt], ta0, sem_a)
            load_bg(p0 + 1)
            wait_a(ta1)
            wait_bg()
            do_pair(p0 + 1, ta1, pending)
            load_bg(nxt)
            pending[0][0].wait()
            pending[0][1].wait()
            pending[1][0].wait()
            pending[1][1].wait()
            return carry

        lax.fori_loop(0, PPT // 2, pairstep, 0)
        wait_a(ta0)
        wait_bg()

    return sc


def kernel(alpha, beta, gamma, alpha_idx, beta_idx, gamma_idx):
    B, N = alpha.shape
    K = alpha_idx.shape[0]
    pai = _pack_idx(alpha_idx)
    pbi = _pack_idx(beta_idx)
    pgi = _pack_idx(gamma_idx)
    a2 = alpha.reshape(B // 2, 2 * N)
    b2 = beta.reshape(B // 2, 2 * N)
    g2 = gamma.reshape(B // 2, 2 * N)
    out_ref = jax.new_ref(jax.lax.empty((B, K), jnp.float32))
    for sl in range(_NSLICES):
        ta, tb, tg = _pack_all(a2, b2, g2, sl)
        _sc_gather(B, N, K, sl)(ta, tb, tg, pai, pbi, pgi, out_ref)
    return jax.freeze(out_ref)


# speedup vs baseline: 1.3205x; 1.2858x over previous
import functools

import jax
import jax.numpy as jnp
from jax import lax
from jax.experimental import pallas as pl
from jax.experimental.pallas import tpu as pltpu
from jax.experimental.pallas import tpu_sc as plsc

_VERY_SMALL = 1e-8


def _not_gamma(gamma):
    B, N = gamma.shape
    blk = 64

    def body(g_ref, o_ref):
        g = g_ref[...]
        o_ref[...] = jnp.log(jnp.maximum(1.0 - jnp.exp(g), _VERY_SMALL))

    return pl.pallas_call(
        body,
        grid=(B // blk,),
        in_specs=[pl.BlockSpec((blk, N), lambda i: (i, 0))],
        out_specs=pl.BlockSpec((blk, N), lambda i: (i, 0)),
        out_shape=jax.ShapeDtypeStruct((B, N), jnp.float32),
    )(gamma)


def _pack_idx(idx):
    r = idx.astype(jnp.int32).reshape(-1, 2, 16)
    return (r[:, 0, :] | (r[:, 1, :] << 16)).reshape(-1)


@functools.cache
def _sc_gather(B, N, K):
    NC, NS = 2, 16
    NW = NC * NS
    RPT = B // NW
    OCH = 4096
    NCH = K // OCH
    GRP = OCH // 32
    KP = K // 2
    assert B % NW == 0 and RPT % 2 == 0 and K % OCH == 0 and K % 32 == 0

    mesh = plsc.VectorSubcoreMesh(core_axis_name="c", subcore_axis_name="s")

    @functools.partial(
        pl.kernel,
        mesh=mesh,
        compiler_params=pltpu.CompilerParams(needs_layout_passes=False),
        out_type=jax.ShapeDtypeStruct((B, K), jnp.float32),
        scratch_types=[
            pltpu.VMEM((KP,), jnp.int32),
            pltpu.VMEM((KP,), jnp.int32),
            pltpu.VMEM((KP,), jnp.int32),
            pltpu.VMEM((N,), jnp.float32),
            pltpu.VMEM((N,), jnp.float32),
            pltpu.VMEM((N,), jnp.float32),
            pltpu.VMEM((N,), jnp.float32),
            pltpu.VMEM((OCH,), jnp.float32),
            pltpu.VMEM((OCH,), jnp.float32),
            pltpu.SemaphoreType.DMA,
            pltpu.SemaphoreType.DMA,
            pltpu.SemaphoreType.DMA,
            pltpu.SemaphoreType.DMA,
        ],
    )
    def sc(a_hbm, b_hbm, g_hbm, pai_hbm, pbi_hbm, pgi_hbm, out_hbm,
           pai, pbi, pgi, arow0, arow1, brow, grow, ob0, ob1,
           sem_in, sem_a, sem_o0, sem_o1):
        wid = lax.axis_index("s") * NC + lax.axis_index("c")
        pltpu.sync_copy(pai_hbm, pai)
        pltpu.sync_copy(pbi_hbm, pbi)
        pltpu.sync_copy(pgi_hbm, pgi)
        row0 = wid * RPT
        m16 = jnp.int32(0xFFFF)

        def wait_a(buf):
            pltpu.make_async_copy(a_hbm.at[0], buf, sem_a).wait()

        def wait_bg():
            pltpu.make_async_copy(b_hbm.at[0], brow, sem_in).wait()
            pltpu.make_async_copy(g_hbm.at[0], grow, sem_in).wait()

        def load_bg(r):
            pltpu.async_copy(b_hbm.at[r], brow, sem_in)
            pltpu.async_copy(g_hbm.at[r], grow, sem_in)

        def do_chunk(c, ob, atab):
            wbase = c * (OCH // 2)

            @plsc.parallel_loop(0, GRP, unroll=4)
            def grp(g):
                w = wbase + g * 16
                wa = pai[pl.ds(w, 16)]
                wb = pbi[pl.ds(w, 16)]
                wg = pgi[pl.ds(w, 16)]
                alo = plsc.load_gather(atab, [lax.bitwise_and(wa, m16)])
                ahi = plsc.load_gather(atab, [lax.shift_right_logical(wa, 16)])
                blo = plsc.load_gather(brow, [lax.bitwise_and(wb, m16)])
                bhi = plsc.load_gather(brow, [lax.shift_right_logical(wb, 16)])
                glo = plsc.load_gather(grow, [lax.bitwise_and(wg, m16)])
                ghi = plsc.load_gather(grow, [lax.shift_right_logical(wg, 16)])
                o = g * 32
                ob[pl.ds(o, 16)] = jnp.maximum(alo + blo - glo, 0.0)
                ob[pl.ds(o + 16, 16)] = jnp.maximum(ahi + bhi - ghi, 0.0)

        def do_row(r, atab, pending):
            for c in range(NCH):
                ob, slot, sem = (ob0, 0, sem_o0) if c % 2 == 0 else (ob1, 1, sem_o1)
                if pending[slot] is not None:
                    pending[slot].wait()
                do_chunk(c, ob, atab)
                pending[slot] = pltpu.async_copy(
                    ob, out_hbm.at[r, pl.ds(c * OCH, OCH)], sem)

        pltpu.async_copy(a_hbm.at[row0], arow0, sem_a)
        load_bg(row0)

        def pair(i, carry):
            r0 = row0 + 2 * i
            pending = [None, None]
            pltpu.async_copy(a_hbm.at[r0 + 1], arow1, sem_a)
            wait_a(arow0)
            wait_bg()
            do_row(r0, arow0, pending)
            nxt = jnp.minimum(r0 + 2, row0 + RPT - 1)
            pltpu.async_copy(a_hbm.at[nxt], arow0, sem_a)
            load_bg(r0 + 1)
            wait_a(arow1)
            wait_bg()
            do_row(r0 + 1, arow1, pending)
            load_bg(nxt)
            pending[0].wait()
            pending[1].wait()
            return carry

        lax.fori_loop(0, RPT // 2, pair, 0)
        wait_a(arow0)
        wait_bg()

    return sc


def kernel(alpha, beta, gamma, alpha_idx, beta_idx, gamma_idx):
    B, N = alpha.shape
    K = alpha_idx.shape[0]
    ng = _not_gamma(gamma)
    pai = _pack_idx(alpha_idx)
    pbi = _pack_idx(beta_idx)
    pgi = _pack_idx(gamma_idx)
    return _sc_gather(B, N, K)(alpha, beta, ng, pai, pbi, pgi)
